# Initial kernel scaffold; baseline (speedup 1.0000x reference)
import jax
import jax.numpy as jnp
from jax.experimental import pallas as pl


def kernel(points):
    raise NotImplementedError("write your pallas kernel here")


# speedup vs baseline: 7.2162x; 7.2162x over previous
import jax
import jax.numpy as jnp
from jax.experimental import pallas as pl
from jax.experimental.pallas import tpu as pltpu

_N = 4096
_D = 64
_B = 512
_NB = _N // _B
_EPS2 = 121.0
_MINPTS = 10.0
_ITERS = 64
_SENT = float(_N)
_BIG = float(_N + 1)


def _dbscan_body(pts_ref, ptst_ref, sqc_ref, sqr_ref, out_ref,
                 adj_ref, degc_ref, vec_a_ref, vec_b_ref, row_a_ref):
    ptst = ptst_ref[...]
    sqr = sqr_ref[...]

    def build(ib, deg_row):
        pb = pts_ref[pl.ds(ib * _B, _B), :]
        g = jax.lax.dot_general(pb, ptst, (((1,), (0,)), ((), ())),
                                preferred_element_type=jnp.float32)
        sqc_b = sqc_ref[pl.ds(ib * _B, _B), :]
        d2 = (sqc_b + sqr) - 2.0 * g
        adj = d2 < _EPS2
        adj_ref[pl.ds(ib * _B, _B), :] = adj.astype(jnp.int8)
        adjf = adj.astype(jnp.float32)
        degc_ref[pl.ds(ib * _B, _B), :] = jnp.sum(adjf, axis=1, keepdims=True)
        return deg_row + jnp.sum(adjf, axis=0, keepdims=True)

    deg_row = jax.lax.fori_loop(0, _NB, build, jnp.zeros((1, _N), jnp.float32))

    core_row = deg_row >= _MINPTS
    core_col = degc_ref[...] >= _MINPTS

    iota_row = jax.lax.broadcasted_iota(jnp.int32, (1, _N), 1).astype(jnp.float32)
    iota_col = jax.lax.broadcasted_iota(jnp.int32, (_N, 1), 0).astype(jnp.float32)

    comp_row0 = jnp.where(core_row, iota_row, _SENT)
    comp_col0 = jnp.where(core_col, iota_col, _SENT)

    def cond(c):
        it, _, _, changed = c
        return jnp.logical_and(changed, it < _ITERS)

    def sweep(c):
        it, comp_row, comp_col, _ = c
        compc_row = jnp.where(core_row, comp_row, _BIG)
        vec_a_ref[...] = jnp.where(core_col, comp_col, _BIG)

        def blk(ib, neigh_row):
            ab = adj_ref[pl.ds(ib * _B, _B), :]
            af = ab.astype(jnp.float32)
            cc_b = vec_a_ref[pl.ds(ib * _B, _B), :]
            neigh_row = jnp.minimum(
                neigh_row,
                jnp.min(_BIG + af * (cc_b - _BIG), axis=0, keepdims=True))
            vec_b_ref[pl.ds(ib * _B, _B), :] = jnp.min(
                _BIG + af * (compc_row - _BIG), axis=1, keepdims=True)
            return neigh_row

        neigh_row = jax.lax.fori_loop(
            0, _NB, blk, jnp.full((1, _N), _BIG, jnp.float32))
        comp_row_n = jnp.minimum(comp_row, neigh_row)
        comp_col_n = jnp.minimum(comp_col, vec_b_ref[...])
        delta = jnp.where(jnp.logical_and(core_row, comp_row_n < comp_row),
                          1.0, 0.0)
        changed = jnp.max(delta) > 0.0
        return it + 1, comp_row_n, comp_col_n, changed

    _, comp_row, comp_col, _ = jax.lax.while_loop(
        cond, sweep, (jnp.int32(0), comp_row0, comp_col0, jnp.bool_(True)))

    is_rep_row = jnp.where(
        jnp.logical_and(core_row, comp_row == iota_row), 1.0, 0.0)
    row_a_ref[...] = is_rep_row
    vec_a_ref[...] = jnp.where(
        jnp.logical_and(core_col, comp_col == iota_col), 1.0, 0.0)

    def crow(jb, acc):
        ir_b = vec_a_ref[pl.ds(jb * _B, _B), :]
        jidx = (jax.lax.broadcasted_iota(jnp.int32, (_B, 1), 0)
                + jb * _B).astype(jnp.float32)
        mask = (jidx <= comp_row).astype(jnp.float32)
        return acc + jnp.sum(mask * ir_b, axis=0, keepdims=True)

    cluster_row = jax.lax.fori_loop(
        0, _NB, crow, jnp.zeros((1, _N), jnp.float32)) - 1.0

    def ccol(jb, acc):
        ir_b = row_a_ref[:, pl.ds(jb * _B, _B)]
        jidx = (jax.lax.broadcasted_iota(jnp.int32, (1, _B), 1)
                + jb * _B).astype(jnp.float32)
        mask = (jidx <= comp_col).astype(jnp.float32)
        return acc + jnp.sum(mask * ir_b, axis=1, keepdims=True)

    cluster_col = jax.lax.fori_loop(
        0, _NB, ccol, jnp.zeros((_N, 1), jnp.float32)) - 1.0
    vec_b_ref[...] = cluster_col

    clc_row = jnp.where(core_row, cluster_row, _BIG)

    def fin(ib, carry):
        af = adj_ref[pl.ds(ib * _B, _B), :].astype(jnp.float32)
        m_b = jnp.min(_BIG + af * (clc_row - _BIG), axis=1, keepdims=True)
        cl_b = vec_b_ref[pl.ds(ib * _B, _B), :]
        kc_b = degc_ref[pl.ds(ib * _B, _B), :] >= _MINPTS
        out_ref[pl.ds(ib * _B, _B), :] = jnp.where(
            kc_b, cl_b, jnp.where(m_b < _BIG, m_b, -1.0))
        return carry

    jax.lax.fori_loop(0, _NB, fin, 0)


def kernel(points):
    pts = points.astype(jnp.float32)
    sq = jnp.sum(pts * pts, axis=1, keepdims=True)
    return pl.pallas_call(
        _dbscan_body,
        out_shape=jax.ShapeDtypeStruct((_N, 1), jnp.float32),
        scratch_shapes=[
            pltpu.VMEM((_N, _N), jnp.int8),
            pltpu.VMEM((_N, 1), jnp.float32),
            pltpu.VMEM((_N, 1), jnp.float32),
            pltpu.VMEM((_N, 1), jnp.float32),
            pltpu.VMEM((1, _N), jnp.float32),
        ],
    )(pts, pts.T, sq, sq.T)
